# baseline (speedup 1.0000x reference)
import functools

import jax
import jax.numpy as jnp
from jax import lax
from jax.experimental import pallas as pl
from jax.experimental.pallas import tpu as pltpu
from jax.experimental.pallas import tpu_sc as plsc

_NC = 2
_NS = 16
_NW = _NC * _NS
_CH = 128


def _tree_sum(vs):
    while len(vs) > 1:
        nxt = [vs[i] + vs[i + 1] for i in range(0, len(vs) - 1, 2)]
        if len(vs) % 2:
            nxt.append(vs[-1])
        vs = nxt
    return vs[0]


@functools.partial(jax.jit, static_argnames=("n", "h", "w"))
def _avg_pool(x2d, n, h, w):
    hw = h * w
    spw = n // _NW
    nch = spw // _CH
    n_valid = (h - 1) * (h - 2)
    inv = 1.0 / float(n_valid)

    mesh = plsc.VectorSubcoreMesh(core_axis_name="c", subcore_axis_name="s")

    @functools.partial(
        pl.kernel,
        out_type=jax.ShapeDtypeStruct((n,), jnp.float32),
        mesh=mesh,
        scratch_types=[
            pltpu.VMEM((_CH, hw), jnp.float32),
            pltpu.VMEM((_CH, hw), jnp.float32),
            pltpu.VMEM((spw,), jnp.float32),
            pltpu.SemaphoreType.DMA,
            pltpu.SemaphoreType.DMA,
        ],
    )
    def sc_kernel(x_hbm, out_hbm, buf0, buf1, outbuf, sem0, sem1):
        wid = lax.axis_index("s") * _NC + lax.axis_index("c")
        base = wid * spw

        lane = lax.iota(jnp.int32, 16)
        wvec = jnp.where((lane >= 1) & (lane <= 14), inv, 0.0).astype(
            jnp.float32
        )
        perms = {k: lane ^ k for k in (1, 2, 4, 8)}
        sels = {k: (lane & k) != 0 for k in (1, 2, 4, 8)}

        def merge(a, b, k):
            pa = a + a.at[perms[k]].get(mode="promise_in_bounds")
            pb = b + b.at[perms[k]].get(mode="promise_in_bounds")
            return jnp.where(sels[k], pb, pa)

        def start(ci, buf, sem):
            pltpu.async_copy(x_hbm.at[pl.ds(base + ci * _CH, _CH)], buf, sem)

        def wait(ci, buf, sem):
            pltpu.make_async_copy(
                x_hbm.at[pl.ds(base + ci * _CH, _CH)], buf, sem
            ).wait()

        def compute(buf, ci):
            @pl.loop(0, _CH // 16)
            def _grp(gi):
                accs = []
                for jj in range(16):
                    j = gi * 16 + jj
                    rows = [buf[j, pl.ds(r * 16, 16)] for r in range(15)]
                    accs.append(_tree_sum(rows) * wvec)
                vs = accs
                for k in (1, 2, 4, 8):
                    vs = [
                        merge(vs[2 * i], vs[2 * i + 1], k)
                        for i in range(len(vs) // 2)
                    ]
                outbuf[pl.ds(ci * _CH + gi * 16, 16)] = vs[0]

        start(0, buf0, sem0)

        @pl.loop(0, nch, step=2)
        def _chunk(ci):
            start(ci + 1, buf1, sem1)
            wait(ci, buf0, sem0)
            compute(buf0, ci)

            @pl.when(ci + 2 < nch)
            def _():
                start(ci + 2, buf0, sem0)

            wait(ci + 1, buf1, sem1)
            compute(buf1, ci + 1)

        pltpu.sync_copy(outbuf, out_hbm.at[pl.ds(base, spw)])

    return sc_kernel(x2d)


def kernel(x):
    b, c, t, h, w = x.shape
    n = b * c * t
    assert h == 16 and w == 16, "kernel specialized to 16x16 spatial grids"
    assert n % (_NW * _CH) == 0 and (n // _NW) % (2 * _CH) == 0
    x2d = x.reshape(n, h * w)
    out = _avg_pool(x2d, n, h, w)
    return out.reshape(b, c, t)
